# Initial kernel scaffold; baseline (speedup 1.0000x reference)
import jax
import jax.numpy as jnp
from jax.experimental import pallas as pl


def kernel(h, coord, coord_diff, edge_attr, W1, b1, W2, b2, W3, edge_index):
    raise NotImplementedError("write your pallas kernel here")


# speedup vs baseline: 2.6531x; 2.6531x over previous
import functools

import jax
import jax.numpy as jnp
from jax import lax
from jax.experimental import pallas as pl
from jax.experimental.pallas import tpu as pltpu
from jax.experimental.pallas import tpu_sc as plsc

N = 10000
E = 320000
H = 128
NORM = 100.0

NHI = 80
NPAD = NHI * 128

NC = 2
NS = 16
NW = NC * NS
EPW = E // NW
BLK = 80

EB = 512
GRID = E // EB


@functools.lru_cache(maxsize=None)
def _sc_mesh():
    return plsc.VectorSubcoreMesh(core_axis_name="c", subcore_axis_name="s",
                                  num_cores=NC, num_subcores=NS)


def _silu(x):
    return x * jax.nn.sigmoid(x)


def _proj_body(h_ref, wa_ref, wb_ref, b1_ref, a_ref, b_ref):
    h = h_ref[...]
    a_ref[...] = jnp.dot(h, wa_ref[...], preferred_element_type=jnp.float32)
    b_ref[...] = jnp.dot(h, wb_ref[...], preferred_element_type=jnp.float32) \
        + b1_ref[...]


def _node_proj(h, Wa, Wb, b1row):
    nb = 2000
    return pl.pallas_call(
        _proj_body,
        grid=(N // nb,),
        in_specs=[
            pl.BlockSpec((nb, H), lambda i: (i, 0)),
            pl.BlockSpec((H, H), lambda i: (0, 0)),
            pl.BlockSpec((H, H), lambda i: (0, 0)),
            pl.BlockSpec((1, H), lambda i: (0, 0)),
        ],
        out_specs=[
            pl.BlockSpec((nb, H), lambda i: (i, 0)),
            pl.BlockSpec((nb, H), lambda i: (i, 0)),
        ],
        out_shape=[
            jax.ShapeDtypeStruct((N, H), jnp.float32),
            jax.ShapeDtypeStruct((N, H), jnp.float32),
        ],
    )(h, Wa, Wb, b1row)


def _gather_body(a_hbm, b_hbm, row_hbm, col_hbm, pre_a_hbm, pre_b_hbm,
                 idxr, idxc, bufa, bufb, sema, semb):
    cid = lax.axis_index("c")
    sid = lax.axis_index("s")
    wid = sid * NC + cid
    base = wid * EPW
    pltpu.sync_copy(row_hbm.at[pl.ds(base, EPW)], idxr)
    pltpu.sync_copy(col_hbm.at[pl.ds(base, EPW)], idxc)

    def step(t, carry):
        off = t * BLK
        cpa = pltpu.async_copy(a_hbm.at[idxr.at[pl.ds(off, BLK)]], bufa, sema)
        cpb = pltpu.async_copy(b_hbm.at[idxc.at[pl.ds(off, BLK)]], bufb, semb)
        cpa.wait()
        cpb.wait()
        pltpu.sync_copy(bufa, pre_a_hbm.at[pl.ds(base + off, BLK)])
        pltpu.sync_copy(bufb, pre_b_hbm.at[pl.ds(base + off, BLK)])
        return carry

    lax.fori_loop(0, EPW // BLK, step, 0)


def _edge_gather(A, B, row, col):
    return pl.kernel(
        _gather_body,
        out_type=[
            jax.ShapeDtypeStruct((E, H), jnp.float32),
            jax.ShapeDtypeStruct((E, H), jnp.float32),
        ],
        mesh=_sc_mesh(),
        scratch_types=[
            pltpu.VMEM((EPW,), jnp.int32),
            pltpu.VMEM((EPW,), jnp.int32),
            pltpu.VMEM((BLK, H), jnp.float32),
            pltpu.VMEM((BLK, H), jnp.float32),
            pltpu.SemaphoreType.DMA,
            pltpu.SemaphoreType.DMA,
        ],
    )(A, B, row, col)


def _mlp_body(pa_ref, pb_ref, attr_ref, cd_ref, row_ref, w2t_ref, b2_ref,
              w1c_ref, w3_ref, coordt_ref, out_ref):
    i = pl.program_id(0)

    pre = pa_ref[...] + pb_ref[...] + attr_ref[...] * w1c_ref[...]
    x1 = _silu(pre)
    x2 = _silu(jnp.dot(x1, w2t_ref[...], preferred_element_type=jnp.float32)
               + b2_ref[...])
    m = jnp.sum(x2 * w3_ref[...], axis=1, keepdims=True)

    row = row_ref[...]
    hi = lax.shift_right_logical(row, 7)
    lo = row & 127
    lanes = lax.broadcasted_iota(jnp.int32, (EB, 128), 1)
    his = lax.broadcasted_iota(jnp.int32, (EB, NHI), 1)
    oh_lo = jnp.where(lanes == lo, 1.0, 0.0)
    oh_hi = jnp.where(his == hi, 1.0, 0.0)

    @pl.when(i == 0)
    def _():
        out_ref[...] = coordt_ref[...] * NORM

    trans = cd_ref[...] * m
    for c in range(3):
        scaled = oh_lo * trans[:, c:c + 1]
        out_ref[c, :, :] += lax.dot_general(
            oh_hi, scaled, (((0,), (0,)), ((), ())),
            preferred_element_type=jnp.float32)

    @pl.when(i == GRID - 1)
    def _():
        out_ref[...] = out_ref[...] * (1.0 / NORM)


def _edge_mlp_agg(preA, preB, attr, cd, row2d, W2T, b2row, w1crow, w3row,
                  coordt):
    return pl.pallas_call(
        _mlp_body,
        grid=(GRID,),
        in_specs=[
            pl.BlockSpec((EB, H), lambda i: (i, 0)),
            pl.BlockSpec((EB, H), lambda i: (i, 0)),
            pl.BlockSpec((EB, 1), lambda i: (i, 0)),
            pl.BlockSpec((EB, 3), lambda i: (i, 0)),
            pl.BlockSpec((EB, 1), lambda i: (i, 0)),
            pl.BlockSpec((H, H), lambda i: (0, 0)),
            pl.BlockSpec((1, H), lambda i: (0, 0)),
            pl.BlockSpec((1, H), lambda i: (0, 0)),
            pl.BlockSpec((1, H), lambda i: (0, 0)),
            pl.BlockSpec((3, NHI, 128), lambda i: (0, 0, 0)),
        ],
        out_specs=pl.BlockSpec((3, NHI, 128), lambda i: (0, 0, 0)),
        out_shape=jax.ShapeDtypeStruct((3, NHI, 128), jnp.float32),
    )(preA, preB, attr, cd, row2d, W2T, b2row, w1crow, w3row, coordt)


@jax.jit
def kernel(h, coord, coord_diff, edge_attr, W1, b1, W2, b2, W3, edge_index):
    W1T = W1.T
    Wa = W1T[:H, :]
    Wb = W1T[H:2 * H, :]
    w1c = W1T[2 * H, :].reshape(1, H)
    b1row = b1.reshape(1, H)
    b2row = b2.reshape(1, H)
    w3row = W3.reshape(1, H)
    W2T = W2.T

    row = edge_index[0]
    col = edge_index[1]
    row2d = row.reshape(E, 1)
    coordt = jnp.concatenate(
        [coord.T, jnp.zeros((3, NPAD - N), jnp.float32)], axis=1
    ).reshape(3, NHI, 128)

    A, B = _node_proj(h, Wa, Wb, b1row)
    preA, preB = _edge_gather(A, B, row, col)
    outt = _edge_mlp_agg(preA, preB, edge_attr, coord_diff, row2d, W2T,
                         b2row, w1c, w3row, coordt)
    return outt.reshape(3, NPAD)[:, :N].T


# speedup vs baseline: 2.6548x; 1.0006x over previous
import functools

import jax
import jax.numpy as jnp
from jax import lax
from jax.experimental import pallas as pl
from jax.experimental.pallas import tpu as pltpu
from jax.experimental.pallas import tpu_sc as plsc

N = 10000
E = 320000
H = 128
NORM = 100.0

NHI = 80
NPAD = NHI * 128

NC = 2
NS = 16
NW = NC * NS
EPW = E // NW
BLK = 80

EB = 512
GRID = E // EB


@functools.lru_cache(maxsize=None)
def _sc_mesh():
    return plsc.VectorSubcoreMesh(core_axis_name="c", subcore_axis_name="s",
                                  num_cores=NC, num_subcores=NS)


def _silu(x):
    return x * jax.nn.sigmoid(x)


def _proj_body(h_ref, wa_ref, wb_ref, b1_ref, a_ref, b_ref):
    h = h_ref[...]
    a_ref[...] = jnp.dot(h, wa_ref[...], preferred_element_type=jnp.float32)
    b_ref[...] = jnp.dot(h, wb_ref[...], preferred_element_type=jnp.float32) \
        + b1_ref[...]


def _node_proj(h, Wa, Wb, b1row):
    nb = 2000
    return pl.pallas_call(
        _proj_body,
        grid=(N // nb,),
        in_specs=[
            pl.BlockSpec((nb, H), lambda i: (i, 0)),
            pl.BlockSpec((H, H), lambda i: (0, 0)),
            pl.BlockSpec((H, H), lambda i: (0, 0)),
            pl.BlockSpec((1, H), lambda i: (0, 0)),
        ],
        out_specs=[
            pl.BlockSpec((nb, H), lambda i: (i, 0)),
            pl.BlockSpec((nb, H), lambda i: (i, 0)),
        ],
        out_shape=[
            jax.ShapeDtypeStruct((N, H), jnp.float32),
            jax.ShapeDtypeStruct((N, H), jnp.float32),
        ],
    )(h, Wa, Wb, b1row)


def _gather_body(a_hbm, b_hbm, row_hbm, col_hbm, pre_a_hbm, pre_b_hbm,
                 idxr, idxc, bufa0, bufb0, bufa1, bufb1,
                 sema0, semb0, sema1, semb1):
    cid = lax.axis_index("c")
    sid = lax.axis_index("s")
    wid = sid * NC + cid
    base = wid * EPW
    pltpu.sync_copy(row_hbm.at[pl.ds(base, EPW)], idxr)
    pltpu.sync_copy(col_hbm.at[pl.ds(base, EPW)], idxc)

    def issue(t, bufa, bufb, sa, sb):
        off = t * BLK
        pltpu.async_copy(a_hbm.at[idxr.at[pl.ds(off, BLK)]], bufa, sa)
        pltpu.async_copy(b_hbm.at[idxc.at[pl.ds(off, BLK)]], bufb, sb)

    def finish(t, bufa, bufb, sa, sb):
        off = t * BLK
        pltpu.make_async_copy(
            a_hbm.at[idxr.at[pl.ds(off, BLK)]], bufa, sa).wait()
        pltpu.make_async_copy(
            b_hbm.at[idxc.at[pl.ds(off, BLK)]], bufb, sb).wait()
        pltpu.sync_copy(bufa, pre_a_hbm.at[pl.ds(base + off, BLK)])
        pltpu.sync_copy(bufb, pre_b_hbm.at[pl.ds(base + off, BLK)])

    nblk = EPW // BLK
    issue(0, bufa0, bufb0, sema0, semb0)

    def step(k, carry):
        issue(2 * k + 1, bufa1, bufb1, sema1, semb1)
        finish(2 * k, bufa0, bufb0, sema0, semb0)
        issue(2 * k + 2, bufa0, bufb0, sema0, semb0)
        finish(2 * k + 1, bufa1, bufb1, sema1, semb1)
        return carry

    lax.fori_loop(0, (nblk - 1) // 2, step, 0)
    finish(nblk - 1, bufa0, bufb0, sema0, semb0)


def _edge_gather(A, B, row, col):
    return pl.kernel(
        _gather_body,
        out_type=[
            jax.ShapeDtypeStruct((E, H), jnp.float32),
            jax.ShapeDtypeStruct((E, H), jnp.float32),
        ],
        mesh=_sc_mesh(),
        scratch_types=[
            pltpu.VMEM((EPW,), jnp.int32),
            pltpu.VMEM((EPW,), jnp.int32),
            pltpu.VMEM((BLK, H), jnp.float32),
            pltpu.VMEM((BLK, H), jnp.float32),
            pltpu.VMEM((BLK, H), jnp.float32),
            pltpu.VMEM((BLK, H), jnp.float32),
            pltpu.SemaphoreType.DMA,
            pltpu.SemaphoreType.DMA,
            pltpu.SemaphoreType.DMA,
            pltpu.SemaphoreType.DMA,
        ],
    )(A, B, row, col)


def _mlp_body(pa_ref, pb_ref, attr_ref, cd_ref, row_ref, w2t_ref, b2_ref,
              w1c_ref, w3_ref, coordt_ref, out_ref):
    i = pl.program_id(0)

    pre = pa_ref[...] + pb_ref[...] + attr_ref[...] * w1c_ref[...]
    x1 = _silu(pre)
    x2 = _silu(jnp.dot(x1, w2t_ref[...], preferred_element_type=jnp.float32)
               + b2_ref[...])
    m = jnp.sum(x2 * w3_ref[...], axis=1, keepdims=True)

    row = row_ref[...]
    hi = lax.shift_right_logical(row, 7)
    lo = row & 127
    lanes = lax.broadcasted_iota(jnp.int32, (EB, 128), 1)
    his = lax.broadcasted_iota(jnp.int32, (EB, NHI), 1)
    oh_lo = jnp.where(lanes == lo, 1.0, 0.0)
    oh_hi = jnp.where(his == hi, 1.0, 0.0)

    @pl.when(i == 0)
    def _():
        out_ref[...] = coordt_ref[...] * NORM

    trans = cd_ref[...] * m
    for c in range(3):
        scaled = oh_lo * trans[:, c:c + 1]
        out_ref[c, :, :] += lax.dot_general(
            oh_hi, scaled, (((0,), (0,)), ((), ())),
            preferred_element_type=jnp.float32)

    @pl.when(i == GRID - 1)
    def _():
        out_ref[...] = out_ref[...] * (1.0 / NORM)


def _edge_mlp_agg(preA, preB, attr, cd, row2d, W2T, b2row, w1crow, w3row,
                  coordt):
    return pl.pallas_call(
        _mlp_body,
        grid=(GRID,),
        in_specs=[
            pl.BlockSpec((EB, H), lambda i: (i, 0)),
            pl.BlockSpec((EB, H), lambda i: (i, 0)),
            pl.BlockSpec((EB, 1), lambda i: (i, 0)),
            pl.BlockSpec((EB, 3), lambda i: (i, 0)),
            pl.BlockSpec((EB, 1), lambda i: (i, 0)),
            pl.BlockSpec((H, H), lambda i: (0, 0)),
            pl.BlockSpec((1, H), lambda i: (0, 0)),
            pl.BlockSpec((1, H), lambda i: (0, 0)),
            pl.BlockSpec((1, H), lambda i: (0, 0)),
            pl.BlockSpec((3, NHI, 128), lambda i: (0, 0, 0)),
        ],
        out_specs=pl.BlockSpec((3, NHI, 128), lambda i: (0, 0, 0)),
        out_shape=jax.ShapeDtypeStruct((3, NHI, 128), jnp.float32),
    )(preA, preB, attr, cd, row2d, W2T, b2row, w1crow, w3row, coordt)


@jax.jit
def kernel(h, coord, coord_diff, edge_attr, W1, b1, W2, b2, W3, edge_index):
    W1T = W1.T
    Wa = W1T[:H, :]
    Wb = W1T[H:2 * H, :]
    w1c = W1T[2 * H, :].reshape(1, H)
    b1row = b1.reshape(1, H)
    b2row = b2.reshape(1, H)
    w3row = W3.reshape(1, H)
    W2T = W2.T

    row = edge_index[0]
    col = edge_index[1]
    row2d = row.reshape(E, 1)
    coordt = jnp.concatenate(
        [coord.T, jnp.zeros((3, NPAD - N), jnp.float32)], axis=1
    ).reshape(3, NHI, 128)

    A, B = _node_proj(h, Wa, Wb, b1row)
    preA, preB = _edge_gather(A, B, row, col)
    outt = _edge_mlp_agg(preA, preB, edge_attr, coord_diff, row2d, W2T,
                         b2row, w1c, w3row, coordt)
    return outt.reshape(3, NPAD)[:, :N].T


# speedup vs baseline: 2.6810x; 1.0099x over previous
import functools

import jax
import jax.numpy as jnp
from jax import lax
from jax.experimental import pallas as pl
from jax.experimental.pallas import tpu as pltpu
from jax.experimental.pallas import tpu_sc as plsc

N = 10000
E = 320000
H = 128
NORM = 100.0

NHI = 80
NPAD = NHI * 128

NC = 2
NS = 16
NW = NC * NS
EPW = E // NW
BLK = 80

EB = 512
GRID = E // EB


@functools.lru_cache(maxsize=None)
def _sc_mesh():
    return plsc.VectorSubcoreMesh(core_axis_name="c", subcore_axis_name="s",
                                  num_cores=NC, num_subcores=NS)


def _silu(x):
    return x * jax.nn.sigmoid(x)


def _proj_body(h_ref, wa_ref, wb_ref, b1_ref, a_ref, b_ref):
    h = h_ref[...]
    a_ref[...] = jnp.dot(h, wa_ref[...], preferred_element_type=jnp.float32)
    b_ref[...] = jnp.dot(h, wb_ref[...], preferred_element_type=jnp.float32) \
        + b1_ref[...]


def _node_proj(h, Wa, Wb, b1row):
    nb = 2000
    return pl.pallas_call(
        _proj_body,
        grid=(N // nb,),
        in_specs=[
            pl.BlockSpec((nb, H), lambda i: (i, 0)),
            pl.BlockSpec((H, H), lambda i: (0, 0)),
            pl.BlockSpec((H, H), lambda i: (0, 0)),
            pl.BlockSpec((1, H), lambda i: (0, 0)),
        ],
        out_specs=[
            pl.BlockSpec((nb, H), lambda i: (i, 0)),
            pl.BlockSpec((nb, H), lambda i: (i, 0)),
        ],
        out_shape=[
            jax.ShapeDtypeStruct((N, H), jnp.float32),
            jax.ShapeDtypeStruct((N, H), jnp.float32),
        ],
    )(h, Wa, Wb, b1row)


def _gather_body(a_hbm, b_hbm, row_hbm, col_hbm, pre_a_hbm, pre_b_hbm,
                 idxr, idxc, bufa0, bufb0, bufa1, bufb1,
                 sema0, semb0, sema1, semb1):
    cid = lax.axis_index("c")
    sid = lax.axis_index("s")
    wid = sid * NC + cid
    base = wid * EPW
    pltpu.sync_copy(row_hbm.at[pl.ds(base, EPW)], idxr)
    pltpu.sync_copy(col_hbm.at[pl.ds(base, EPW)], idxc)

    def issue(t, bufa, bufb, sa, sb):
        off = t * BLK
        pltpu.async_copy(a_hbm.at[idxr.at[pl.ds(off, BLK)]], bufa, sa)
        pltpu.async_copy(b_hbm.at[idxc.at[pl.ds(off, BLK)]], bufb, sb)

    def finish(t, bufa, bufb, sa, sb):
        off = t * BLK
        pltpu.make_async_copy(
            a_hbm.at[idxr.at[pl.ds(off, BLK)]], bufa, sa).wait()
        pltpu.make_async_copy(
            b_hbm.at[idxc.at[pl.ds(off, BLK)]], bufb, sb).wait()
        pltpu.sync_copy(bufa, pre_a_hbm.at[pl.ds(base + off, BLK)])
        pltpu.sync_copy(bufb, pre_b_hbm.at[pl.ds(base + off, BLK)])

    nblk = EPW // BLK
    issue(0, bufa0, bufb0, sema0, semb0)

    def step(k, carry):
        issue(2 * k + 1, bufa1, bufb1, sema1, semb1)
        finish(2 * k, bufa0, bufb0, sema0, semb0)
        issue(2 * k + 2, bufa0, bufb0, sema0, semb0)
        finish(2 * k + 1, bufa1, bufb1, sema1, semb1)
        return carry

    lax.fori_loop(0, (nblk - 1) // 2, step, 0)
    finish(nblk - 1, bufa0, bufb0, sema0, semb0)


def _edge_gather(A, B, row, col):
    return pl.kernel(
        _gather_body,
        out_type=[
            jax.ShapeDtypeStruct((E, H), jnp.float32),
            jax.ShapeDtypeStruct((E, H), jnp.float32),
        ],
        mesh=_sc_mesh(),
        scratch_types=[
            pltpu.VMEM((EPW,), jnp.int32),
            pltpu.VMEM((EPW,), jnp.int32),
            pltpu.VMEM((BLK, H), jnp.float32),
            pltpu.VMEM((BLK, H), jnp.float32),
            pltpu.VMEM((BLK, H), jnp.float32),
            pltpu.VMEM((BLK, H), jnp.float32),
            pltpu.SemaphoreType.DMA,
            pltpu.SemaphoreType.DMA,
            pltpu.SemaphoreType.DMA,
            pltpu.SemaphoreType.DMA,
        ],
    )(A, B, row, col)


def _mlp_body(pa_ref, pb_ref, attr_ref, cd_ref, row_ref, rowt_ref, w2t_ref,
              b2_ref, w1c_ref, w3_ref, coordt_ref, out_ref):
    i = pl.program_id(0)

    pre = pa_ref[...] + pb_ref[...] + attr_ref[...] * w1c_ref[...]
    x1 = _silu(pre)
    x2 = _silu(jnp.dot(x1.astype(jnp.bfloat16), w2t_ref[...],
                       preferred_element_type=jnp.float32) + b2_ref[...])
    m = jnp.sum(x2 * w3_ref[...], axis=1, keepdims=True)

    row = row_ref[...]
    lo = row & 127
    lanes = lax.broadcasted_iota(jnp.int32, (EB, 128), 1)
    oh_lo = jnp.where(lanes == lo, 1.0, 0.0)

    hit = lax.shift_right_logical(rowt_ref[...], 7)
    hrows = lax.broadcasted_iota(jnp.int32, (NHI, EB), 0)
    oh_hit = jnp.where(hrows == hit, 1.0, 0.0).astype(jnp.bfloat16)

    @pl.when(i == 0)
    def _():
        out_ref[...] = coordt_ref[...] * NORM

    trans = cd_ref[...] * m
    for c in range(3):
        scaled = (oh_lo * trans[:, c:c + 1]).astype(jnp.bfloat16)
        out_ref[c, :, :] += jnp.dot(oh_hit, scaled,
                                    preferred_element_type=jnp.float32)

    @pl.when(i == GRID - 1)
    def _():
        out_ref[...] = out_ref[...] * (1.0 / NORM)


def _edge_mlp_agg(preA, preB, attr, cd, row2d, rowt2d, W2T, b2row, w1crow,
                  w3row, coordt):
    return pl.pallas_call(
        _mlp_body,
        grid=(GRID,),
        in_specs=[
            pl.BlockSpec((EB, H), lambda i: (i, 0)),
            pl.BlockSpec((EB, H), lambda i: (i, 0)),
            pl.BlockSpec((EB, 1), lambda i: (i, 0)),
            pl.BlockSpec((EB, 3), lambda i: (i, 0)),
            pl.BlockSpec((EB, 1), lambda i: (i, 0)),
            pl.BlockSpec((1, EB), lambda i: (0, i)),
            pl.BlockSpec((H, H), lambda i: (0, 0)),
            pl.BlockSpec((1, H), lambda i: (0, 0)),
            pl.BlockSpec((1, H), lambda i: (0, 0)),
            pl.BlockSpec((1, H), lambda i: (0, 0)),
            pl.BlockSpec((3, NHI, 128), lambda i: (0, 0, 0)),
        ],
        out_specs=pl.BlockSpec((3, NHI, 128), lambda i: (0, 0, 0)),
        out_shape=jax.ShapeDtypeStruct((3, NHI, 128), jnp.float32),
    )(preA, preB, attr, cd, row2d, rowt2d, W2T, b2row, w1crow, w3row, coordt)


@jax.jit
def kernel(h, coord, coord_diff, edge_attr, W1, b1, W2, b2, W3, edge_index):
    W1T = W1.T
    Wa = W1T[:H, :]
    Wb = W1T[H:2 * H, :]
    w1c = W1T[2 * H, :].reshape(1, H)
    b1row = b1.reshape(1, H)
    b2row = b2.reshape(1, H)
    w3row = W3.reshape(1, H)
    W2T = W2.T

    row = edge_index[0]
    col = edge_index[1]
    row2d = row.reshape(E, 1)
    rowt2d = row.reshape(1, E)
    coordt = jnp.concatenate(
        [coord.T, jnp.zeros((3, NPAD - N), jnp.float32)], axis=1
    ).reshape(3, NHI, 128)

    A, B = _node_proj(h, Wa, Wb, b1row)
    preA, preB = _edge_gather(A, B, row, col)
    outt = _edge_mlp_agg(preA, preB, edge_attr, coord_diff, row2d, rowt2d,
                         W2T.astype(jnp.bfloat16), b2row, w1c, w3row, coordt)
    return outt.reshape(3, NPAD)[:, :N].T


# speedup vs baseline: 3.3095x; 1.2344x over previous
import functools

import jax
import jax.numpy as jnp
from jax import lax
from jax.experimental import pallas as pl
from jax.experimental.pallas import tpu as pltpu
from jax.experimental.pallas import tpu_sc as plsc

N = 10000
E = 320000
H = 128
NORM = 100.0

NHI = 80
NPAD = NHI * 128

NC = 2
NS = 16
NW = NC * NS
EPW = E // NW
BLK = 80

EB = 1024
GRID = E // EB


@functools.lru_cache(maxsize=None)
def _sc_mesh():
    return plsc.VectorSubcoreMesh(core_axis_name="c", subcore_axis_name="s",
                                  num_cores=NC, num_subcores=NS)


def _silu(x):
    return x * jax.nn.sigmoid(x)


def _proj_body(h_ref, wa_ref, wb_ref, b1_ref, a_ref, b_ref):
    h = h_ref[...]
    a_ref[...] = jnp.dot(h, wa_ref[...], preferred_element_type=jnp.float32)
    b_ref[...] = jnp.dot(h, wb_ref[...], preferred_element_type=jnp.float32) \
        + b1_ref[...]


def _node_proj(h, Wa, Wb, b1row):
    nb = 2000
    return pl.pallas_call(
        _proj_body,
        grid=(N // nb,),
        in_specs=[
            pl.BlockSpec((nb, H), lambda i: (i, 0)),
            pl.BlockSpec((H, H), lambda i: (0, 0)),
            pl.BlockSpec((H, H), lambda i: (0, 0)),
            pl.BlockSpec((1, H), lambda i: (0, 0)),
        ],
        out_specs=[
            pl.BlockSpec((nb, H), lambda i: (i, 0)),
            pl.BlockSpec((nb, H), lambda i: (i, 0)),
        ],
        out_shape=[
            jax.ShapeDtypeStruct((N, H), jnp.float32),
            jax.ShapeDtypeStruct((N, H), jnp.float32),
        ],
    )(h, Wa, Wb, b1row)


def _gather_body(a_hbm, b_hbm, row_hbm, col_hbm, pre_a_hbm, pre_b_hbm,
                 idxr, idxc, bufa0, bufb0, bufa1, bufb1,
                 sema0, semb0, sema1, semb1):
    cid = lax.axis_index("c")
    sid = lax.axis_index("s")
    wid = sid * NC + cid
    base = wid * EPW
    pltpu.sync_copy(row_hbm.at[pl.ds(base, EPW)], idxr)
    pltpu.sync_copy(col_hbm.at[pl.ds(base, EPW)], idxc)

    def issue(t, bufa, bufb, sa, sb):
        off = t * BLK
        pltpu.async_copy(a_hbm.at[idxr.at[pl.ds(off, BLK)]], bufa, sa)
        pltpu.async_copy(b_hbm.at[idxc.at[pl.ds(off, BLK)]], bufb, sb)

    def finish(t, bufa, bufb, sa, sb):
        off = t * BLK
        pltpu.make_async_copy(
            a_hbm.at[idxr.at[pl.ds(off, BLK)]], bufa, sa).wait()
        pltpu.make_async_copy(
            b_hbm.at[idxc.at[pl.ds(off, BLK)]], bufb, sb).wait()
        pltpu.sync_copy(bufa, pre_a_hbm.at[pl.ds(base + off, BLK)])
        pltpu.sync_copy(bufb, pre_b_hbm.at[pl.ds(base + off, BLK)])

    nblk = EPW // BLK
    issue(0, bufa0, bufb0, sema0, semb0)

    def step(k, carry):
        issue(2 * k + 1, bufa1, bufb1, sema1, semb1)
        finish(2 * k, bufa0, bufb0, sema0, semb0)
        issue(2 * k + 2, bufa0, bufb0, sema0, semb0)
        finish(2 * k + 1, bufa1, bufb1, sema1, semb1)
        return carry

    lax.fori_loop(0, (nblk - 1) // 2, step, 0)
    finish(nblk - 1, bufa0, bufb0, sema0, semb0)


def _edge_gather(A, B, row, col):
    return pl.kernel(
        _gather_body,
        out_type=[
            jax.ShapeDtypeStruct((E, H), jnp.float32),
            jax.ShapeDtypeStruct((E, H), jnp.float32),
        ],
        mesh=_sc_mesh(),
        scratch_types=[
            pltpu.VMEM((EPW,), jnp.int32),
            pltpu.VMEM((EPW,), jnp.int32),
            pltpu.VMEM((BLK, H), jnp.float32),
            pltpu.VMEM((BLK, H), jnp.float32),
            pltpu.VMEM((BLK, H), jnp.float32),
            pltpu.VMEM((BLK, H), jnp.float32),
            pltpu.SemaphoreType.DMA,
            pltpu.SemaphoreType.DMA,
            pltpu.SemaphoreType.DMA,
            pltpu.SemaphoreType.DMA,
        ],
    )(A, B, row, col)


def _mlp_body(pa_ref, pb_ref, attr_ref, cd_ref, row_ref, rowt_ref, w2t_ref,
              b2_ref, w1c_ref, w3_ref, coordt_ref, out_ref):
    i = pl.program_id(0)

    pre = pa_ref[...] + pb_ref[...] + attr_ref[...] * w1c_ref[...]
    x1 = _silu(pre)
    x2 = _silu(jnp.dot(x1.astype(jnp.bfloat16), w2t_ref[...],
                       preferred_element_type=jnp.float32) + b2_ref[...])
    m = jnp.sum(x2 * w3_ref[...], axis=1, keepdims=True)

    row = row_ref[...]
    lo = row & 127
    lanes = lax.broadcasted_iota(jnp.int32, (EB, 128), 1)
    oh_lo = jnp.where(lanes == lo, 1.0, 0.0)

    hit = lax.shift_right_logical(rowt_ref[...], 7)
    hrows = lax.broadcasted_iota(jnp.int32, (NHI, EB), 0)
    oh_hit = jnp.where(hrows == hit, 1.0, 0.0).astype(jnp.bfloat16)

    @pl.when(i == 0)
    def _():
        out_ref[...] = coordt_ref[...] * NORM

    trans = cd_ref[...] * m
    for c in range(3):
        scaled = (oh_lo * trans[:, c:c + 1]).astype(jnp.bfloat16)
        out_ref[c, :, :] += jnp.dot(oh_hit, scaled,
                                    preferred_element_type=jnp.float32)

    @pl.when(i == GRID - 1)
    def _():
        out_ref[...] = out_ref[...] * (1.0 / NORM)


def _edge_mlp_agg(preA, preB, attr, cd, row2d, rowt2d, W2T, b2row, w1crow,
                  w3row, coordt):
    return pl.pallas_call(
        _mlp_body,
        grid=(GRID,),
        in_specs=[
            pl.BlockSpec((EB, H), lambda i: (i, 0)),
            pl.BlockSpec((EB, H), lambda i: (i, 0)),
            pl.BlockSpec((EB, 1), lambda i: (i, 0)),
            pl.BlockSpec((EB, 3), lambda i: (i, 0)),
            pl.BlockSpec((EB, 1), lambda i: (i, 0)),
            pl.BlockSpec((1, EB), lambda i: (0, i)),
            pl.BlockSpec((H, H), lambda i: (0, 0)),
            pl.BlockSpec((1, H), lambda i: (0, 0)),
            pl.BlockSpec((1, H), lambda i: (0, 0)),
            pl.BlockSpec((1, H), lambda i: (0, 0)),
            pl.BlockSpec((3, NHI, 128), lambda i: (0, 0, 0)),
        ],
        out_specs=pl.BlockSpec((3, NHI, 128), lambda i: (0, 0, 0)),
        out_shape=jax.ShapeDtypeStruct((3, NHI, 128), jnp.float32),
    )(preA, preB, attr, cd, row2d, rowt2d, W2T, b2row, w1crow, w3row, coordt)


@jax.jit
def kernel(h, coord, coord_diff, edge_attr, W1, b1, W2, b2, W3, edge_index):
    W1T = W1.T
    Wa = W1T[:H, :]
    Wb = W1T[H:2 * H, :]
    w1c = W1T[2 * H, :].reshape(1, H)
    b1row = b1.reshape(1, H)
    b2row = b2.reshape(1, H)
    w3row = W3.reshape(1, H)
    W2T = W2.T

    row = edge_index[0]
    col = edge_index[1]
    row2d = row.reshape(E, 1)
    rowt2d = row.reshape(1, E)
    coordt = jnp.concatenate(
        [coord.T, jnp.zeros((3, NPAD - N), jnp.float32)], axis=1
    ).reshape(3, NHI, 128)

    A, B = _node_proj(h, Wa, Wb, b1row)
    preA, preB = _edge_gather(A, B, row, col)
    outt = _edge_mlp_agg(preA, preB, edge_attr, coord_diff, row2d, rowt2d,
                         W2T.astype(jnp.bfloat16), b2row, w1c, w3row, coordt)
    return outt.reshape(3, NPAD)[:, :N].T


# speedup vs baseline: 3.4501x; 1.0425x over previous
import functools

import jax
import jax.numpy as jnp
from jax import lax
from jax.experimental import pallas as pl
from jax.experimental.pallas import tpu as pltpu
from jax.experimental.pallas import tpu_sc as plsc

N = 10000
E = 320000
H = 128
NORM = 100.0

NHI = 80
NPAD = NHI * 128

NC = 2
NS = 16
NW = NC * NS
EPW = E // NW
BLK = 80

EB = 1280
GRID = E // EB


@functools.lru_cache(maxsize=None)
def _sc_mesh():
    return plsc.VectorSubcoreMesh(core_axis_name="c", subcore_axis_name="s",
                                  num_cores=NC, num_subcores=NS)


def _silu(x):
    return x * jax.nn.sigmoid(x)


def _proj_body(h_ref, wa_ref, wb_ref, b1_ref, a_ref, b_ref):
    h = h_ref[...]
    a_ref[...] = jnp.dot(h, wa_ref[...], preferred_element_type=jnp.float32)
    b_ref[...] = jnp.dot(h, wb_ref[...], preferred_element_type=jnp.float32) \
        + b1_ref[...]


def _node_proj(h, Wa, Wb, b1row):
    nb = 2000
    return pl.pallas_call(
        _proj_body,
        grid=(N // nb,),
        in_specs=[
            pl.BlockSpec((nb, H), lambda i: (i, 0)),
            pl.BlockSpec((H, H), lambda i: (0, 0)),
            pl.BlockSpec((H, H), lambda i: (0, 0)),
            pl.BlockSpec((1, H), lambda i: (0, 0)),
        ],
        out_specs=[
            pl.BlockSpec((nb, H), lambda i: (i, 0)),
            pl.BlockSpec((nb, H), lambda i: (i, 0)),
        ],
        out_shape=[
            jax.ShapeDtypeStruct((N, H), jnp.float32),
            jax.ShapeDtypeStruct((N, H), jnp.float32),
        ],
    )(h, Wa, Wb, b1row)


def _gather_body(a_hbm, b_hbm, row_hbm, col_hbm, pre_a_hbm, pre_b_hbm,
                 idxr, idxc, bufa0, bufb0, bufa1, bufb1,
                 sema0, semb0, sema1, semb1):
    cid = lax.axis_index("c")
    sid = lax.axis_index("s")
    wid = sid * NC + cid
    base = wid * EPW
    pltpu.sync_copy(row_hbm.at[pl.ds(base, EPW)], idxr)
    pltpu.sync_copy(col_hbm.at[pl.ds(base, EPW)], idxc)

    def issue(t, bufa, bufb, sa, sb):
        off = t * BLK
        pltpu.async_copy(a_hbm.at[idxr.at[pl.ds(off, BLK)]], bufa, sa)
        pltpu.async_copy(b_hbm.at[idxc.at[pl.ds(off, BLK)]], bufb, sb)

    def finish(t, bufa, bufb, sa, sb):
        off = t * BLK
        pltpu.make_async_copy(
            a_hbm.at[idxr.at[pl.ds(off, BLK)]], bufa, sa).wait()
        pltpu.make_async_copy(
            b_hbm.at[idxc.at[pl.ds(off, BLK)]], bufb, sb).wait()
        pltpu.sync_copy(bufa, pre_a_hbm.at[pl.ds(base + off, BLK)])
        pltpu.sync_copy(bufb, pre_b_hbm.at[pl.ds(base + off, BLK)])

    nblk = EPW // BLK
    issue(0, bufa0, bufb0, sema0, semb0)

    def step(k, carry):
        issue(2 * k + 1, bufa1, bufb1, sema1, semb1)
        finish(2 * k, bufa0, bufb0, sema0, semb0)
        issue(2 * k + 2, bufa0, bufb0, sema0, semb0)
        finish(2 * k + 1, bufa1, bufb1, sema1, semb1)
        return carry

    lax.fori_loop(0, (nblk - 1) // 2, step, 0)
    finish(nblk - 1, bufa0, bufb0, sema0, semb0)


def _edge_gather(A, B, row, col):
    return pl.kernel(
        _gather_body,
        out_type=[
            jax.ShapeDtypeStruct((E, H), jnp.float32),
            jax.ShapeDtypeStruct((E, H), jnp.float32),
        ],
        mesh=_sc_mesh(),
        scratch_types=[
            pltpu.VMEM((EPW,), jnp.int32),
            pltpu.VMEM((EPW,), jnp.int32),
            pltpu.VMEM((BLK, H), jnp.float32),
            pltpu.VMEM((BLK, H), jnp.float32),
            pltpu.VMEM((BLK, H), jnp.float32),
            pltpu.VMEM((BLK, H), jnp.float32),
            pltpu.SemaphoreType.DMA,
            pltpu.SemaphoreType.DMA,
            pltpu.SemaphoreType.DMA,
            pltpu.SemaphoreType.DMA,
        ],
    )(A, B, row, col)


def _mlp_body(pa_ref, pb_ref, attr_ref, cd_ref, row_ref, rowt_ref, w2t_ref,
              b2_ref, w1c_ref, w3_ref, coordt_ref, out_ref):
    i = pl.program_id(0)

    pre = pa_ref[...] + pb_ref[...] + attr_ref[...] * w1c_ref[...]
    x1 = _silu(pre)
    x2 = _silu(jnp.dot(x1.astype(jnp.bfloat16), w2t_ref[...],
                       preferred_element_type=jnp.float32) + b2_ref[...])
    m = jnp.sum(x2 * w3_ref[...], axis=1, keepdims=True)

    row = row_ref[...]
    lo = row & 127
    lanes = lax.broadcasted_iota(jnp.int32, (EB, 128), 1)
    oh_lo = jnp.where(lanes == lo, 1.0, 0.0)

    hit = lax.shift_right_logical(rowt_ref[...], 7)
    hrows = lax.broadcasted_iota(jnp.int32, (NHI, EB), 0)
    oh_hit = jnp.where(hrows == hit, 1.0, 0.0).astype(jnp.bfloat16)

    @pl.when(i == 0)
    def _():
        out_ref[...] = coordt_ref[...] * NORM

    trans = cd_ref[...] * m
    for c in range(3):
        scaled = (oh_lo * trans[:, c:c + 1]).astype(jnp.bfloat16)
        out_ref[c, :, :] += jnp.dot(oh_hit, scaled,
                                    preferred_element_type=jnp.float32)

    @pl.when(i == GRID - 1)
    def _():
        out_ref[...] = out_ref[...] * (1.0 / NORM)


def _edge_mlp_agg(preA, preB, attr, cd, row2d, rowt2d, W2T, b2row, w1crow,
                  w3row, coordt):
    return pl.pallas_call(
        _mlp_body,
        grid=(GRID,),
        in_specs=[
            pl.BlockSpec((EB, H), lambda i: (i, 0)),
            pl.BlockSpec((EB, H), lambda i: (i, 0)),
            pl.BlockSpec((EB, 1), lambda i: (i, 0)),
            pl.BlockSpec((EB, 3), lambda i: (i, 0)),
            pl.BlockSpec((EB, 1), lambda i: (i, 0)),
            pl.BlockSpec((1, EB), lambda i: (0, i)),
            pl.BlockSpec((H, H), lambda i: (0, 0)),
            pl.BlockSpec((1, H), lambda i: (0, 0)),
            pl.BlockSpec((1, H), lambda i: (0, 0)),
            pl.BlockSpec((1, H), lambda i: (0, 0)),
            pl.BlockSpec((3, NHI, 128), lambda i: (0, 0, 0)),
        ],
        out_specs=pl.BlockSpec((3, NHI, 128), lambda i: (0, 0, 0)),
        out_shape=jax.ShapeDtypeStruct((3, NHI, 128), jnp.float32),
    )(preA, preB, attr, cd, row2d, rowt2d, W2T, b2row, w1crow, w3row, coordt)


@jax.jit
def kernel(h, coord, coord_diff, edge_attr, W1, b1, W2, b2, W3, edge_index):
    W1T = W1.T
    Wa = W1T[:H, :]
    Wb = W1T[H:2 * H, :]
    w1c = W1T[2 * H, :].reshape(1, H)
    b1row = b1.reshape(1, H)
    b2row = b2.reshape(1, H)
    w3row = W3.reshape(1, H)
    W2T = W2.T

    row = edge_index[0]
    col = edge_index[1]
    row2d = row.reshape(E, 1)
    rowt2d = row.reshape(1, E)
    coordt = jnp.concatenate(
        [coord.T, jnp.zeros((3, NPAD - N), jnp.float32)], axis=1
    ).reshape(3, NHI, 128)

    A, B = _node_proj(h, Wa, Wb, b1row)
    preA, preB = _edge_gather(A, B, row, col)
    outt = _edge_mlp_agg(preA, preB, edge_attr, coord_diff, row2d, rowt2d,
                         W2T.astype(jnp.bfloat16), b2row, w1c, w3row, coordt)
    return outt.reshape(3, NPAD)[:, :N].T


# speedup vs baseline: 3.8568x; 1.1179x over previous
import functools

import jax
import jax.numpy as jnp
from jax import lax
from jax.experimental import pallas as pl
from jax.experimental.pallas import tpu as pltpu
from jax.experimental.pallas import tpu_sc as plsc

N = 10000
E = 320000
H = 128
NORM = 100.0

NHI = 80
NPAD = NHI * 128

NC = 2
NS = 16
NW = NC * NS
EPW = E // NW
BLK = 80

EB = 1280
GRID = E // EB


@functools.lru_cache(maxsize=None)
def _sc_mesh():
    return plsc.VectorSubcoreMesh(core_axis_name="c", subcore_axis_name="s",
                                  num_cores=NC, num_subcores=NS)


def _silu(x):
    return x * jax.nn.sigmoid(x)


def _proj_body(h_ref, wa_ref, wb_ref, b1_ref, a_ref, b_ref):
    h = h_ref[...]
    a_ref[...] = jnp.dot(h, wa_ref[...], preferred_element_type=jnp.float32)
    b_ref[...] = jnp.dot(h, wb_ref[...], preferred_element_type=jnp.float32) \
        + b1_ref[...]


def _node_proj(h, Wa, Wb, b1row):
    nb = 2000
    return pl.pallas_call(
        _proj_body,
        grid=(N // nb,),
        in_specs=[
            pl.BlockSpec((nb, H), lambda i: (i, 0)),
            pl.BlockSpec((H, H), lambda i: (0, 0)),
            pl.BlockSpec((H, H), lambda i: (0, 0)),
            pl.BlockSpec((1, H), lambda i: (0, 0)),
        ],
        out_specs=[
            pl.BlockSpec((nb, H), lambda i: (i, 0)),
            pl.BlockSpec((nb, H), lambda i: (i, 0)),
        ],
        out_shape=[
            jax.ShapeDtypeStruct((N, H), jnp.float32),
            jax.ShapeDtypeStruct((N, H), jnp.float32),
        ],
    )(h, Wa, Wb, b1row)


def _gather_body(a_hbm, b_hbm, row_hbm, col_hbm, pre_hbm,
                 idxr, idxc, buf0, buf1, sema0, semb0, sema1, semb1):
    cid = lax.axis_index("c")
    sid = lax.axis_index("s")
    wid = sid * NC + cid
    base = wid * EPW
    pltpu.sync_copy(row_hbm.at[pl.ds(base, EPW)], idxr)
    pltpu.sync_copy(col_hbm.at[pl.ds(base, EPW)], idxc)

    def issue_a(t, buf, sa):
        off = t * BLK
        pltpu.async_copy(a_hbm.at[idxr.at[pl.ds(off, BLK)]], buf, sa)

    def finish(t, buf, sa, sb):
        off = t * BLK
        pltpu.make_async_copy(
            a_hbm.at[idxr.at[pl.ds(off, BLK)]], buf, sa).wait()
        pltpu.async_copy(
            b_hbm.at[idxc.at[pl.ds(off, BLK)]], buf, sb, add=True)
        pltpu.make_async_copy(
            b_hbm.at[idxc.at[pl.ds(off, BLK)]], buf, sb).wait()
        pltpu.sync_copy(buf, pre_hbm.at[pl.ds(base + off, BLK)])

    nblk = EPW // BLK
    issue_a(0, buf0, sema0)

    def step(k, carry):
        issue_a(2 * k + 1, buf1, sema1)
        finish(2 * k, buf0, sema0, semb0)
        issue_a(2 * k + 2, buf0, sema0)
        finish(2 * k + 1, buf1, sema1, semb1)
        return carry

    lax.fori_loop(0, (nblk - 1) // 2, step, 0)
    finish(nblk - 1, buf0, sema0, semb0)


def _edge_gather(A, B, row, col):
    return pl.kernel(
        _gather_body,
        out_type=jax.ShapeDtypeStruct((E, H), jnp.float32),
        mesh=_sc_mesh(),
        scratch_types=[
            pltpu.VMEM((EPW,), jnp.int32),
            pltpu.VMEM((EPW,), jnp.int32),
            pltpu.VMEM((BLK, H), jnp.float32),
            pltpu.VMEM((BLK, H), jnp.float32),
            pltpu.SemaphoreType.DMA,
            pltpu.SemaphoreType.DMA,
            pltpu.SemaphoreType.DMA,
            pltpu.SemaphoreType.DMA,
        ],
    )(A, B, row, col)


def _mlp_body(pre_ref, attr_ref, cd_ref, row_ref, rowt_ref, w2t_ref,
              b2_ref, w1c_ref, w3_ref, coordt_ref, out_ref):
    i = pl.program_id(0)

    pre = pre_ref[...] + attr_ref[...] * w1c_ref[...]
    x1 = _silu(pre)
    x2 = _silu(jnp.dot(x1.astype(jnp.bfloat16), w2t_ref[...],
                       preferred_element_type=jnp.float32) + b2_ref[...])
    m = jnp.sum(x2 * w3_ref[...], axis=1, keepdims=True)

    row = row_ref[...]
    lo = row & 127
    lanes = lax.broadcasted_iota(jnp.int32, (EB, 128), 1)
    oh_lo = jnp.where(lanes == lo, 1.0, 0.0)

    hit = lax.shift_right_logical(rowt_ref[...], 7)
    hrows = lax.broadcasted_iota(jnp.int32, (NHI, EB), 0)
    oh_hit = jnp.where(hrows == hit, 1.0, 0.0).astype(jnp.bfloat16)

    @pl.when(i == 0)
    def _():
        out_ref[...] = coordt_ref[...] * NORM

    trans = cd_ref[...] * m
    for c in range(3):
        scaled = (oh_lo * trans[:, c:c + 1]).astype(jnp.bfloat16)
        out_ref[c, :, :] += jnp.dot(oh_hit, scaled,
                                    preferred_element_type=jnp.float32)

    @pl.when(i == GRID - 1)
    def _():
        out_ref[...] = out_ref[...] * (1.0 / NORM)


def _edge_mlp_agg(pre, attr, cd, row2d, rowt2d, W2T, b2row, w1crow,
                  w3row, coordt):
    return pl.pallas_call(
        _mlp_body,
        grid=(GRID,),
        in_specs=[
            pl.BlockSpec((EB, H), lambda i: (i, 0)),
            pl.BlockSpec((EB, 1), lambda i: (i, 0)),
            pl.BlockSpec((EB, 3), lambda i: (i, 0)),
            pl.BlockSpec((EB, 1), lambda i: (i, 0)),
            pl.BlockSpec((1, EB), lambda i: (0, i)),
            pl.BlockSpec((H, H), lambda i: (0, 0)),
            pl.BlockSpec((1, H), lambda i: (0, 0)),
            pl.BlockSpec((1, H), lambda i: (0, 0)),
            pl.BlockSpec((1, H), lambda i: (0, 0)),
            pl.BlockSpec((3, NHI, 128), lambda i: (0, 0, 0)),
        ],
        out_specs=pl.BlockSpec((3, NHI, 128), lambda i: (0, 0, 0)),
        out_shape=jax.ShapeDtypeStruct((3, NHI, 128), jnp.float32),
    )(pre, attr, cd, row2d, rowt2d, W2T, b2row, w1crow, w3row, coordt)


@jax.jit
def kernel(h, coord, coord_diff, edge_attr, W1, b1, W2, b2, W3, edge_index):
    W1T = W1.T
    Wa = W1T[:H, :]
    Wb = W1T[H:2 * H, :]
    w1c = W1T[2 * H, :].reshape(1, H)
    b1row = b1.reshape(1, H)
    b2row = b2.reshape(1, H)
    w3row = W3.reshape(1, H)
    W2T = W2.T

    row = edge_index[0]
    col = edge_index[1]
    row2d = row.reshape(E, 1)
    rowt2d = row.reshape(1, E)
    coordt = jnp.concatenate(
        [coord.T, jnp.zeros((3, NPAD - N), jnp.float32)], axis=1
    ).reshape(3, NHI, 128)

    A, B = _node_proj(h, Wa, Wb, b1row)
    pre = _edge_gather(A, B, row, col)
    outt = _edge_mlp_agg(pre, edge_attr, coord_diff, row2d, rowt2d,
                         W2T.astype(jnp.bfloat16), b2row, w1c, w3row, coordt)
    return outt.reshape(3, NPAD)[:, :N].T


# speedup vs baseline: 4.2520x; 1.1025x over previous
import functools

import jax
import jax.numpy as jnp
from jax import lax
from jax.experimental import pallas as pl
from jax.experimental.pallas import tpu as pltpu
from jax.experimental.pallas import tpu_sc as plsc

N = 10000
E = 320000
H = 128
NORM = 100.0

NHI = 80
NPAD = NHI * 128

NC = 2
NS = 16
NW = NC * NS
EPW = E // NW
BLK = 80

EB = 2560
GRID = E // EB


@functools.lru_cache(maxsize=None)
def _sc_mesh():
    return plsc.VectorSubcoreMesh(core_axis_name="c", subcore_axis_name="s",
                                  num_cores=NC, num_subcores=NS)


def _silu(x):
    return x * jax.nn.sigmoid(x)


def _proj_body(h_ref, wa_ref, wb_ref, b1_ref, a_ref, b_ref):
    h = h_ref[...]
    a_ref[...] = jnp.dot(h, wa_ref[...], preferred_element_type=jnp.float32)
    b_ref[...] = jnp.dot(h, wb_ref[...], preferred_element_type=jnp.float32) \
        + b1_ref[...]


def _node_proj(h, Wa, Wb, b1row):
    nb = 2000
    return pl.pallas_call(
        _proj_body,
        grid=(N // nb,),
        in_specs=[
            pl.BlockSpec((nb, H), lambda i: (i, 0)),
            pl.BlockSpec((H, H), lambda i: (0, 0)),
            pl.BlockSpec((H, H), lambda i: (0, 0)),
            pl.BlockSpec((1, H), lambda i: (0, 0)),
        ],
        out_specs=[
            pl.BlockSpec((nb, H), lambda i: (i, 0)),
            pl.BlockSpec((nb, H), lambda i: (i, 0)),
        ],
        out_shape=[
            jax.ShapeDtypeStruct((N, H), jnp.float32),
            jax.ShapeDtypeStruct((N, H), jnp.float32),
        ],
    )(h, Wa, Wb, b1row)


def _gather_body(a_hbm, b_hbm, row_hbm, col_hbm, pre_hbm,
                 idxr, idxc, buf0, buf1, sema0, semb0, sema1, semb1):
    cid = lax.axis_index("c")
    sid = lax.axis_index("s")
    wid = sid * NC + cid
    base = wid * EPW
    pltpu.sync_copy(row_hbm.at[pl.ds(base, EPW)], idxr)
    pltpu.sync_copy(col_hbm.at[pl.ds(base, EPW)], idxc)

    def issue_a(t, buf, sa):
        off = t * BLK
        pltpu.async_copy(a_hbm.at[idxr.at[pl.ds(off, BLK)]], buf, sa)

    def finish(t, buf, sa, sb):
        off = t * BLK
        pltpu.make_async_copy(
            a_hbm.at[idxr.at[pl.ds(off, BLK)]], buf, sa).wait()
        pltpu.async_copy(
            b_hbm.at[idxc.at[pl.ds(off, BLK)]], buf, sb, add=True)
        pltpu.make_async_copy(
            b_hbm.at[idxc.at[pl.ds(off, BLK)]], buf, sb).wait()
        pltpu.sync_copy(buf, pre_hbm.at[pl.ds(base + off, BLK)])

    nblk = EPW // BLK
    issue_a(0, buf0, sema0)

    def step(k, carry):
        issue_a(2 * k + 1, buf1, sema1)
        finish(2 * k, buf0, sema0, semb0)
        issue_a(2 * k + 2, buf0, sema0)
        finish(2 * k + 1, buf1, sema1, semb1)
        return carry

    lax.fori_loop(0, (nblk - 1) // 2, step, 0)
    finish(nblk - 1, buf0, sema0, semb0)


def _edge_gather(A, B, row, col):
    return pl.kernel(
        _gather_body,
        out_type=jax.ShapeDtypeStruct((E, H), jnp.float32),
        mesh=_sc_mesh(),
        scratch_types=[
            pltpu.VMEM((EPW,), jnp.int32),
            pltpu.VMEM((EPW,), jnp.int32),
            pltpu.VMEM((BLK, H), jnp.float32),
            pltpu.VMEM((BLK, H), jnp.float32),
            pltpu.SemaphoreType.DMA,
            pltpu.SemaphoreType.DMA,
            pltpu.SemaphoreType.DMA,
            pltpu.SemaphoreType.DMA,
        ],
    )(A, B, row, col)


def _mlp_body(pre_ref, attr_ref, cd_ref, row_ref, rowt_ref, w2t_ref,
              b2_ref, w1c_ref, w3_ref, coordt_ref, out_ref):
    i = pl.program_id(0)

    pre = pre_ref[...] + attr_ref[...] * w1c_ref[...]
    x1 = _silu(pre)
    x2 = _silu(jnp.dot(x1.astype(jnp.bfloat16), w2t_ref[...],
                       preferred_element_type=jnp.float32) + b2_ref[...])
    m = jnp.sum(x2 * w3_ref[...], axis=1, keepdims=True)

    row = row_ref[...]
    lo = row & 127
    lanes = lax.broadcasted_iota(jnp.int32, (EB, 128), 1)
    oh_lo = jnp.where(lanes == lo, 1.0, 0.0)

    hit = lax.shift_right_logical(rowt_ref[...], 7)
    hrows = lax.broadcasted_iota(jnp.int32, (NHI, EB), 0)
    oh_hit = jnp.where(hrows == hit, 1.0, 0.0).astype(jnp.bfloat16)

    @pl.when(i == 0)
    def _():
        out_ref[...] = coordt_ref[...] * NORM

    trans = cd_ref[...] * m
    for c in range(3):
        scaled = (oh_lo * trans[:, c:c + 1]).astype(jnp.bfloat16)
        out_ref[c, :, :] += jnp.dot(oh_hit, scaled,
                                    preferred_element_type=jnp.float32)

    @pl.when(i == GRID - 1)
    def _():
        out_ref[...] = out_ref[...] * (1.0 / NORM)


def _edge_mlp_agg(pre, attr, cd, row2d, rowt2d, W2T, b2row, w1crow,
                  w3row, coordt):
    return pl.pallas_call(
        _mlp_body,
        grid=(GRID,),
        in_specs=[
            pl.BlockSpec((EB, H), lambda i: (i, 0)),
            pl.BlockSpec((EB, 1), lambda i: (i, 0)),
            pl.BlockSpec((EB, 3), lambda i: (i, 0)),
            pl.BlockSpec((EB, 1), lambda i: (i, 0)),
            pl.BlockSpec((1, EB), lambda i: (0, i)),
            pl.BlockSpec((H, H), lambda i: (0, 0)),
            pl.BlockSpec((1, H), lambda i: (0, 0)),
            pl.BlockSpec((1, H), lambda i: (0, 0)),
            pl.BlockSpec((1, H), lambda i: (0, 0)),
            pl.BlockSpec((3, NHI, 128), lambda i: (0, 0, 0)),
        ],
        out_specs=pl.BlockSpec((3, NHI, 128), lambda i: (0, 0, 0)),
        out_shape=jax.ShapeDtypeStruct((3, NHI, 128), jnp.float32),
    )(pre, attr, cd, row2d, rowt2d, W2T, b2row, w1crow, w3row, coordt)


@jax.jit
def kernel(h, coord, coord_diff, edge_attr, W1, b1, W2, b2, W3, edge_index):
    W1T = W1.T
    Wa = W1T[:H, :]
    Wb = W1T[H:2 * H, :]
    w1c = W1T[2 * H, :].reshape(1, H)
    b1row = b1.reshape(1, H)
    b2row = b2.reshape(1, H)
    w3row = W3.reshape(1, H)
    W2T = W2.T

    row = edge_index[0]
    col = edge_index[1]
    row2d = row.reshape(E, 1)
    rowt2d = row.reshape(1, E)
    coordt = jnp.concatenate(
        [coord.T, jnp.zeros((3, NPAD - N), jnp.float32)], axis=1
    ).reshape(3, NHI, 128)

    A, B = _node_proj(h, Wa, Wb, b1row)
    pre = _edge_gather(A, B, row, col)
    outt = _edge_mlp_agg(pre, edge_attr, coord_diff, row2d, rowt2d,
                         W2T.astype(jnp.bfloat16), b2row, w1c, w3row, coordt)
    return outt.reshape(3, NPAD)[:, :N].T


# speedup vs baseline: 4.2959x; 1.0103x over previous
import functools

import jax
import jax.numpy as jnp
from jax import lax
from jax.experimental import pallas as pl
from jax.experimental.pallas import tpu as pltpu
from jax.experimental.pallas import tpu_sc as plsc

N = 10000
E = 320000
H = 128
NORM = 100.0

NHI = 80
NPAD = NHI * 128

NC = 2
NS = 16
NW = NC * NS
EPW = E // NW
BLK = 80

EB = 3200
GRID = E // EB


@functools.lru_cache(maxsize=None)
def _sc_mesh():
    return plsc.VectorSubcoreMesh(core_axis_name="c", subcore_axis_name="s",
                                  num_cores=NC, num_subcores=NS)


def _silu(x):
    return x * jax.nn.sigmoid(x)


def _proj_body(h_ref, wa_ref, wb_ref, b1_ref, a_ref, b_ref):
    h = h_ref[...]
    a_ref[...] = jnp.dot(h, wa_ref[...], preferred_element_type=jnp.float32)
    b_ref[...] = jnp.dot(h, wb_ref[...], preferred_element_type=jnp.float32) \
        + b1_ref[...]


def _node_proj(h, Wa, Wb, b1row):
    nb = 2000
    return pl.pallas_call(
        _proj_body,
        grid=(N // nb,),
        in_specs=[
            pl.BlockSpec((nb, H), lambda i: (i, 0)),
            pl.BlockSpec((H, H), lambda i: (0, 0)),
            pl.BlockSpec((H, H), lambda i: (0, 0)),
            pl.BlockSpec((1, H), lambda i: (0, 0)),
        ],
        out_specs=[
            pl.BlockSpec((nb, H), lambda i: (i, 0)),
            pl.BlockSpec((nb, H), lambda i: (i, 0)),
        ],
        out_shape=[
            jax.ShapeDtypeStruct((N, H), jnp.float32),
            jax.ShapeDtypeStruct((N, H), jnp.float32),
        ],
    )(h, Wa, Wb, b1row)


def _gather_body(a_hbm, b_hbm, row_hbm, col_hbm, pre_hbm,
                 idxr, idxc, buf0, buf1, sema0, semb0, sema1, semb1):
    cid = lax.axis_index("c")
    sid = lax.axis_index("s")
    wid = sid * NC + cid
    base = wid * EPW
    pltpu.sync_copy(row_hbm.at[pl.ds(base, EPW)], idxr)
    pltpu.sync_copy(col_hbm.at[pl.ds(base, EPW)], idxc)

    def issue_a(t, buf, sa):
        off = t * BLK
        pltpu.async_copy(a_hbm.at[idxr.at[pl.ds(off, BLK)]], buf, sa)

    def finish(t, buf, sa, sb):
        off = t * BLK
        pltpu.make_async_copy(
            a_hbm.at[idxr.at[pl.ds(off, BLK)]], buf, sa).wait()
        pltpu.async_copy(
            b_hbm.at[idxc.at[pl.ds(off, BLK)]], buf, sb, add=True)
        pltpu.make_async_copy(
            b_hbm.at[idxc.at[pl.ds(off, BLK)]], buf, sb).wait()
        pltpu.sync_copy(buf, pre_hbm.at[pl.ds(base + off, BLK)])

    nblk = EPW // BLK
    issue_a(0, buf0, sema0)

    def step(k, carry):
        issue_a(2 * k + 1, buf1, sema1)
        finish(2 * k, buf0, sema0, semb0)
        issue_a(2 * k + 2, buf0, sema0)
        finish(2 * k + 1, buf1, sema1, semb1)
        return carry

    lax.fori_loop(0, (nblk - 1) // 2, step, 0)
    finish(nblk - 1, buf0, sema0, semb0)


def _edge_gather(A, B, row, col):
    return pl.kernel(
        _gather_body,
        out_type=jax.ShapeDtypeStruct((E, H), jnp.float32),
        mesh=_sc_mesh(),
        scratch_types=[
            pltpu.VMEM((EPW,), jnp.int32),
            pltpu.VMEM((EPW,), jnp.int32),
            pltpu.VMEM((BLK, H), jnp.float32),
            pltpu.VMEM((BLK, H), jnp.float32),
            pltpu.SemaphoreType.DMA,
            pltpu.SemaphoreType.DMA,
            pltpu.SemaphoreType.DMA,
            pltpu.SemaphoreType.DMA,
        ],
    )(A, B, row, col)


def _mlp_body(pre_ref, attr_ref, cd_ref, row_ref, rowt_ref, w2t_ref,
              b2_ref, w1c_ref, w3_ref, coordt_ref, out_ref):
    i = pl.program_id(0)

    pre = pre_ref[...] + attr_ref[...] * w1c_ref[...]
    x1 = _silu(pre)
    x2 = _silu(jnp.dot(x1.astype(jnp.bfloat16), w2t_ref[...],
                       preferred_element_type=jnp.float32) + b2_ref[...])
    m = jnp.sum(x2 * w3_ref[...], axis=1, keepdims=True)

    row = row_ref[...]
    lo = row & 127
    lanes = lax.broadcasted_iota(jnp.int32, (EB, 128), 1)
    oh_lo = jnp.where(lanes == lo, 1.0, 0.0)

    hit = lax.shift_right_logical(rowt_ref[...], 7)
    hrows = lax.broadcasted_iota(jnp.int32, (NHI, EB), 0)
    oh_hit = jnp.where(hrows == hit, 1.0, 0.0).astype(jnp.bfloat16)

    @pl.when(i == 0)
    def _():
        out_ref[...] = coordt_ref[...] * NORM

    trans = cd_ref[...] * m
    for c in range(3):
        scaled = (oh_lo * trans[:, c:c + 1]).astype(jnp.bfloat16)
        out_ref[c, :, :] += jnp.dot(oh_hit, scaled,
                                    preferred_element_type=jnp.float32)

    @pl.when(i == GRID - 1)
    def _():
        out_ref[...] = out_ref[...] * (1.0 / NORM)


def _edge_mlp_agg(pre, attr, cd, row2d, rowt2d, W2T, b2row, w1crow,
                  w3row, coordt):
    return pl.pallas_call(
        _mlp_body,
        grid=(GRID,),
        in_specs=[
            pl.BlockSpec((EB, H), lambda i: (i, 0)),
            pl.BlockSpec((EB, 1), lambda i: (i, 0)),
            pl.BlockSpec((EB, 3), lambda i: (i, 0)),
            pl.BlockSpec((EB, 1), lambda i: (i, 0)),
            pl.BlockSpec((1, EB), lambda i: (0, i)),
            pl.BlockSpec((H, H), lambda i: (0, 0)),
            pl.BlockSpec((1, H), lambda i: (0, 0)),
            pl.BlockSpec((1, H), lambda i: (0, 0)),
            pl.BlockSpec((1, H), lambda i: (0, 0)),
            pl.BlockSpec((3, NHI, 128), lambda i: (0, 0, 0)),
        ],
        out_specs=pl.BlockSpec((3, NHI, 128), lambda i: (0, 0, 0)),
        out_shape=jax.ShapeDtypeStruct((3, NHI, 128), jnp.float32),
    )(pre, attr, cd, row2d, rowt2d, W2T, b2row, w1crow, w3row, coordt)


@jax.jit
def kernel(h, coord, coord_diff, edge_attr, W1, b1, W2, b2, W3, edge_index):
    W1T = W1.T
    Wa = W1T[:H, :]
    Wb = W1T[H:2 * H, :]
    w1c = W1T[2 * H, :].reshape(1, H)
    b1row = b1.reshape(1, H)
    b2row = b2.reshape(1, H)
    w3row = W3.reshape(1, H)
    W2T = W2.T

    row = edge_index[0]
    col = edge_index[1]
    row2d = row.reshape(E, 1)
    rowt2d = row.reshape(1, E)
    coordt = jnp.concatenate(
        [coord.T, jnp.zeros((3, NPAD - N), jnp.float32)], axis=1
    ).reshape(3, NHI, 128)

    A, B = _node_proj(h, Wa, Wb, b1row)
    pre = _edge_gather(A, B, row, col)
    outt = _edge_mlp_agg(pre, edge_attr, coord_diff, row2d, rowt2d,
                         W2T.astype(jnp.bfloat16), b2row, w1c, w3row, coordt)
    return outt.reshape(3, NPAD)[:, :N].T
